# Initial kernel scaffold; baseline (speedup 1.0000x reference)
import functools

import jax
import jax.numpy as jnp
from jax import lax
from jax.experimental import pallas as pl
from jax.experimental.pallas import tpu as pltpu
from jax.experimental.pallas import tpu_sc as plsc

NC, NS, L = 2, 16, 16
NW = NC * NS
B = 16
S = 512
N = 256
H = 768
HH = H // 2
BS = B * S
SEG = B * N
NNZ = 16384
EPS = 1e-12

ROWS_PER_W = BS // NW
A_CHUNK = 32
A_STEPS = ROWS_PER_W // A_CHUNK

NNZ_PER_W = NNZ // NS
B_CHUNK = 64
B_STEPS = NNZ_PER_W // B_CHUNK
SEG_PER_W = SEG // NS

_mesh = plsc.VectorSubcoreMesh(
    core_axis_name="c", subcore_axis_name="s", num_cores=NC, num_subcores=NS
)


@functools.partial(
    pl.kernel,
    out_type=jax.ShapeDtypeStruct((2 * BS, HH), jnp.float32),
    mesh=_mesh,
    scratch_types=[
        pltpu.VMEM((A_CHUNK,), jnp.int32),
        pltpu.VMEM((A_CHUNK,), jnp.int32),
        pltpu.VMEM((A_CHUNK,), jnp.int32),
        pltpu.VMEM((A_CHUNK, H), jnp.float32),
        pltpu.VMEM((A_CHUNK, H), jnp.float32),
        pltpu.VMEM((A_CHUNK, H), jnp.float32),
        pltpu.SemaphoreType.DMA,
        pltpu.SemaphoreType.DMA,
        pltpu.SemaphoreType.DMA,
    ],
)
def _phase_a(ids_hbm, pos_hbm, tt_hbm, wtab, ptab, ttab, flat_out,
             widx, pidx, tidx, wrows, prows, trows, sem0, sem1, sem2):
    wid = lax.axis_index("s") * NC + lax.axis_index("c")
    base = wid * ROWS_PER_W

    @pl.loop(0, A_STEPS)
    def _(step):
        off = base + step * A_CHUNK
        pltpu.sync_copy(ids_hbm.at[pl.ds(off, A_CHUNK)], widx)
        pltpu.sync_copy(pos_hbm.at[pl.ds(off, A_CHUNK)], pidx)
        pltpu.sync_copy(tt_hbm.at[pl.ds(off, A_CHUNK)], tidx)
        cw = pltpu.async_copy(wtab.at[widx], wrows, sem0)
        cp = pltpu.async_copy(ptab.at[pidx], prows, sem1)
        ct = pltpu.async_copy(ttab.at[tidx], trows, sem2)
        cw.wait()
        cp.wait()
        ct.wait()

        @pl.loop(0, A_CHUNK)
        def _(r):
            for c in range(H // L):
                sl = (r, pl.ds(c * L, L))
                wrows.at[*sl][...] = (
                    wrows.at[*sl][...] + prows.at[*sl][...] + trows.at[*sl][...]
                )

        pltpu.sync_copy(wrows.at[pl.ds(0, A_CHUNK), pl.ds(0, HH)],
                        flat_out.at[pl.ds(off, A_CHUNK)])
        pltpu.sync_copy(wrows.at[pl.ds(0, A_CHUNK), pl.ds(HH, HH)],
                        flat_out.at[pl.ds(BS + off, A_CHUNK)])


@functools.partial(
    pl.kernel,
    out_type=jax.ShapeDtypeStruct((2 * SEG, HH), jnp.float32),
    mesh=_mesh,
    scratch_types=[
        pltpu.VMEM((NNZ_PER_W,), jnp.int32),
        pltpu.VMEM((NNZ_PER_W,), jnp.int32),
        pltpu.VMEM((NNZ_PER_W,), jnp.int32),
        pltpu.VMEM((NNZ_PER_W,), jnp.float32),
        pltpu.VMEM((B_STEPS, B_CHUNK), jnp.int32),
        pltpu.VMEM((B_STEPS, B_CHUNK), jnp.int32),
        pltpu.VMEM((B_CHUNK, HH), jnp.float32),
        pltpu.VMEM_SHARED((SEG, HH), jnp.float32),
        pltpu.SemaphoreType.DMA,
    ],
)
def _phase_b(flat_hbm, mask_hbm, vals_hbm, out_hbm,
             tmpb, tmpn, tmps, vals_v, seg2d, src2d, rows, acc, sem):
    cid = lax.axis_index("c")
    sid = lax.axis_index("s")
    nbase = sid * NNZ_PER_W

    pltpu.sync_copy(mask_hbm.at[0, pl.ds(nbase, NNZ_PER_W)], tmpb)
    pltpu.sync_copy(mask_hbm.at[1, pl.ds(nbase, NNZ_PER_W)], tmpn)
    pltpu.sync_copy(mask_hbm.at[2, pl.ds(nbase, NNZ_PER_W)], tmps)
    pltpu.sync_copy(vals_hbm.at[pl.ds(nbase, NNZ_PER_W)], vals_v)

    @pl.loop(0, B_CHUNK)
    def _(r):
        for c in range(HH // L):
            rows.at[r, pl.ds(c * L, L)][...] = jnp.zeros((L,), jnp.float32)

    @pl.loop(0, SEG_PER_W // B_CHUNK)
    def _(z):
        pltpu.sync_copy(rows, acc.at[pl.ds(sid * SEG_PER_W + z * B_CHUNK, B_CHUNK)])

    @pl.loop(0, NNZ_PER_W // L)
    def _(i):
        st = i // (B_CHUNK // L)
        off = (i % (B_CHUNK // L)) * L
        sl = pl.ds(i * L, L)
        bb = tmpb.at[sl][...]
        nn = tmpn.at[sl][...]
        ss = tmps.at[sl][...]
        seg2d.at[st, pl.ds(off, L)][...] = bb * N + nn
        src2d.at[st, pl.ds(off, L)][...] = bb * S + ss + cid * BS

    plsc.subcore_barrier()

    @pl.loop(0, B_STEPS)
    def _(st):
        pltpu.async_copy(flat_hbm.at[src2d.at[st]], rows, sem).wait()

        @pl.loop(0, B_CHUNK)
        def _(r):
            vb = plsc.load_gather(
                vals_v, [jnp.full((L,), st * B_CHUNK + r, jnp.int32)])
            for c in range(HH // L):
                sl = (r, pl.ds(c * L, L))
                rows.at[*sl][...] = rows.at[*sl][...] * vb

        pltpu.sync_copy(rows, acc.at[seg2d.at[st]], add=True)

    plsc.subcore_barrier()

    @pl.loop(0, SEG_PER_W // B_CHUNK)
    def _(z):
        ro = sid * SEG_PER_W + z * B_CHUNK
        pltpu.sync_copy(acc.at[pl.ds(ro, B_CHUNK)],
                        out_hbm.at[pl.ds(cid * SEG + ro, B_CHUNK)])


_LN_R = 256


def _ln_body(x_ref, g_ref, b_ref, o_ref):
    x = jnp.concatenate([x_ref[0], x_ref[1]], axis=-1)
    mu = jnp.mean(x, axis=-1, keepdims=True)
    xc = x - mu
    var = jnp.mean(xc * xc, axis=-1, keepdims=True)
    o_ref[...] = xc * lax.rsqrt(var + EPS) * g_ref[...] + b_ref[...]


def kernel(input_ids, mask_indices, mask_values, position_ids, token_type_ids,
           word_emb, pos_emb, type_emb, ln_gamma, ln_beta):
    ids = jnp.asarray(input_ids, jnp.int32).reshape(BS)
    pos = jnp.asarray(position_ids, jnp.int32).reshape(BS)
    tts = jnp.asarray(token_type_ids, jnp.int32).reshape(BS)
    mask = jnp.asarray(mask_indices, jnp.int32)

    flat = _phase_a(ids, pos, tts, word_emb, pos_emb, type_emb)
    node2 = _phase_b(flat, mask, mask_values)

    out = pl.pallas_call(
        _ln_body,
        grid=(SEG // _LN_R,),
        in_specs=[
            pl.BlockSpec((2, _LN_R, HH), lambda i: (0, i, 0)),
            pl.BlockSpec((1, H), lambda i: (0, 0)),
            pl.BlockSpec((1, H), lambda i: (0, 0)),
        ],
        out_specs=pl.BlockSpec((_LN_R, H), lambda i: (i, 0)),
        out_shape=jax.ShapeDtypeStruct((SEG, H), jnp.float32),
    )(node2.reshape(2, SEG, HH), ln_gamma.reshape(1, H), ln_beta.reshape(1, H))

    return out.reshape(B, N, H)


# speedup vs baseline: 1.3416x; 1.3416x over previous
import dataclasses
import functools

import jax
import jax.numpy as jnp
from jax import lax
from jax.experimental import pallas as pl
from jax.experimental.pallas import tpu as pltpu
from jax.experimental.pallas import tpu_sc as plsc

NC, NS, L = 2, 16, 16
NW = NC * NS
B = 16
S = 512
N = 256
H = 768
W = 128
STRIPS = H // W
SPC = STRIPS // NC
BS = B * S
SEG = B * N
NNZ = 16384
EPS = 1e-12

ROWS_PER_W = BS // NW
A_CHUNK = 32
A_STEPS = ROWS_PER_W // A_CHUNK

NNZ_PER_W = NNZ // NS
B_CHUNK = 64
B_STEPS = NNZ_PER_W // B_CHUNK
ACC_ROWS = SPC * SEG
OUT_PER_W = ACC_ROWS // NS

_mesh = plsc.VectorSubcoreMesh(
    core_axis_name="c", subcore_axis_name="s", num_cores=NC, num_subcores=NS
)

_cp = pltpu.CompilerParams()
if "needs_layout_passes" in pltpu.CompilerParams.__dataclass_fields__:
    _cp = dataclasses.replace(_cp, needs_layout_passes=False)


@functools.partial(
    pl.kernel,
    out_type=jax.ShapeDtypeStruct((STRIPS * BS, W), jnp.float32),
    mesh=_mesh,
    scratch_types=[
        pltpu.VMEM((A_CHUNK,), jnp.int32),
        pltpu.VMEM((A_CHUNK,), jnp.int32),
        pltpu.VMEM((A_CHUNK,), jnp.int32),
        pltpu.VMEM((A_CHUNK, H), jnp.float32),
        pltpu.VMEM((A_CHUNK, H), jnp.float32),
        pltpu.VMEM((A_CHUNK, H), jnp.float32),
        pltpu.SemaphoreType.DMA,
        pltpu.SemaphoreType.DMA,
        pltpu.SemaphoreType.DMA,
    ],
)
def _phase_a(ids_hbm, pos_hbm, tt_hbm, wtab, ptab, ttab, flat_out,
             widx, pidx, tidx, wrows, prows, trows, sem0, sem1, sem2):
    wid = lax.axis_index("s") * NC + lax.axis_index("c")
    base = wid * ROWS_PER_W

    @pl.loop(0, A_STEPS)
    def _(step):
        off = base + step * A_CHUNK
        pltpu.sync_copy(ids_hbm.at[pl.ds(off, A_CHUNK)], widx)
        pltpu.sync_copy(pos_hbm.at[pl.ds(off, A_CHUNK)], pidx)
        pltpu.sync_copy(tt_hbm.at[pl.ds(off, A_CHUNK)], tidx)
        cw = pltpu.async_copy(wtab.at[widx], wrows, sem0)
        cp = pltpu.async_copy(ptab.at[pidx], prows, sem1)
        ct = pltpu.async_copy(ttab.at[tidx], trows, sem2)
        cw.wait()
        cp.wait()
        ct.wait()

        @pl.loop(0, A_CHUNK)
        def _(r):
            for c in range(H // L):
                sl = (r, pl.ds(c * L, L))
                wrows.at[*sl][...] = (
                    wrows.at[*sl][...] + prows.at[*sl][...] + trows.at[*sl][...]
                )

        for k in range(STRIPS):
            pltpu.sync_copy(wrows.at[pl.ds(0, A_CHUNK), pl.ds(k * W, W)],
                            flat_out.at[pl.ds(k * BS + off, A_CHUNK)])


@functools.partial(
    pl.kernel,
    out_type=jax.ShapeDtypeStruct((NC * ACC_ROWS, W), jnp.float32),
    mesh=_mesh,
    scratch_types=[
        pltpu.VMEM((NNZ_PER_W,), jnp.int32),
        pltpu.VMEM((NNZ_PER_W,), jnp.int32),
        pltpu.VMEM((NNZ_PER_W,), jnp.int32),
        pltpu.VMEM((NNZ_PER_W,), jnp.float32),
        [pltpu.VMEM((B_CHUNK,), jnp.int32) for _ in range(SPC)],
        [pltpu.VMEM((B_CHUNK,), jnp.int32) for _ in range(SPC)],
        pltpu.VMEM((B_CHUNK, W), jnp.float32),
        pltpu.VMEM_SHARED((ACC_ROWS, W), jnp.float32),
        pltpu.SemaphoreType.DMA,
    ],
    compiler_params=_cp,
)
def _phase_b(flat_hbm, bidx_hbm, nidx_hbm, sidx_hbm, vals_hbm, out_hbm,
             tmpb, tmpn, tmps, vals_v, segs, srcs, rows, acc, sem):
    cid = lax.axis_index("c")
    sid = lax.axis_index("s")
    nbase = sid * NNZ_PER_W

    pltpu.sync_copy(bidx_hbm.at[pl.ds(nbase, NNZ_PER_W)], tmpb)
    pltpu.sync_copy(nidx_hbm.at[pl.ds(nbase, NNZ_PER_W)], tmpn)
    pltpu.sync_copy(sidx_hbm.at[pl.ds(nbase, NNZ_PER_W)], tmps)
    pltpu.sync_copy(vals_hbm.at[pl.ds(nbase, NNZ_PER_W)], vals_v)

    @pl.loop(0, B_CHUNK)
    def _(r):
        for c in range(W // L):
            rows.at[r, pl.ds(c * L, L)][...] = jnp.zeros((L,), jnp.float32)

    @pl.loop(0, OUT_PER_W // B_CHUNK)
    def _(z):
        pltpu.sync_copy(rows, acc.at[pl.ds(sid * OUT_PER_W + z * B_CHUNK, B_CHUNK)])

    plsc.subcore_barrier()

    @pl.loop(0, B_STEPS)
    def _(st):
        for c in range(B_CHUNK // L):
            sl = pl.ds(st * B_CHUNK + c * L, L)
            dst = pl.ds(c * L, L)
            bb = tmpb.at[sl][...]
            seg = bb * N + tmpn.at[sl][...]
            src = bb * S + tmps.at[sl][...] + cid * (SPC * BS)
            for k in range(SPC):
                segs[k].at[dst][...] = seg + k * SEG
                srcs[k].at[dst][...] = src + k * BS

        for k in range(SPC):
            pltpu.async_copy(flat_hbm.at[srcs[k]], rows, sem).wait()

            @pl.loop(0, B_CHUNK)
            def _(r):
                vb = plsc.load_gather(
                    vals_v, [jnp.full((L,), st * B_CHUNK + r, jnp.int32)])
                for c in range(W // L):
                    sl = (r, pl.ds(c * L, L))
                    rows.at[*sl][...] = rows.at[*sl][...] * vb

            pltpu.sync_copy(rows, acc.at[segs[k]], add=True)

    plsc.subcore_barrier()

    @pl.loop(0, OUT_PER_W // B_CHUNK)
    def _(z):
        ro = sid * OUT_PER_W + z * B_CHUNK
        pltpu.sync_copy(acc.at[pl.ds(ro, B_CHUNK)],
                        out_hbm.at[pl.ds(cid * ACC_ROWS + ro, B_CHUNK)])


_LN_R = 256


def _ln_body(x_ref, g_ref, b_ref, o_ref):
    x = jnp.concatenate([x_ref[k] for k in range(STRIPS)], axis=-1)
    mu = jnp.mean(x, axis=-1, keepdims=True)
    xc = x - mu
    var = jnp.mean(xc * xc, axis=-1, keepdims=True)
    o_ref[...] = xc * lax.rsqrt(var + EPS) * g_ref[...] + b_ref[...]


def kernel(input_ids, mask_indices, mask_values, position_ids, token_type_ids,
           word_emb, pos_emb, type_emb, ln_gamma, ln_beta):
    ids = jnp.asarray(input_ids, jnp.int32).reshape(BS)
    pos = jnp.asarray(position_ids, jnp.int32).reshape(BS)
    tts = jnp.asarray(token_type_ids, jnp.int32).reshape(BS)
    mask = jnp.asarray(mask_indices, jnp.int32)

    flat = _phase_a(ids, pos, tts, word_emb, pos_emb, type_emb)
    node6 = _phase_b(flat, mask[0], mask[1], mask[2], mask_values)

    out = pl.pallas_call(
        _ln_body,
        grid=(SEG // _LN_R,),
        in_specs=[
            pl.BlockSpec((STRIPS, _LN_R, W), lambda i: (0, i, 0)),
            pl.BlockSpec((1, H), lambda i: (0, 0)),
            pl.BlockSpec((1, H), lambda i: (0, 0)),
        ],
        out_specs=pl.BlockSpec((_LN_R, H), lambda i: (i, 0)),
        out_shape=jax.ShapeDtypeStruct((SEG, H), jnp.float32),
    )(node6.reshape(STRIPS, SEG, W), ln_gamma.reshape(1, H), ln_beta.reshape(1, H))

    return out.reshape(B, N, H)


# speedup vs baseline: 2.5413x; 1.8943x over previous
import dataclasses
import functools

import jax
import jax.numpy as jnp
from jax import lax
from jax.experimental import pallas as pl
from jax.experimental.pallas import tpu as pltpu
from jax.experimental.pallas import tpu_sc as plsc

NC, NS, L = 2, 16, 16
NW = NC * NS
B = 16
S = 512
N = 256
H = 768
W = 128
STRIPS = H // W
SPC = STRIPS // NC
BS = B * S
SEG = B * N
NNZ = 16384
EPS = 1e-12

ROWS_PER_W = BS // NW
A_CHUNK = 32
A_STEPS = ROWS_PER_W // A_CHUNK

NNZ_PER_W = NNZ // NS
B_CHUNK = 64
B_STEPS = NNZ_PER_W // B_CHUNK
B_ITEMS = B_STEPS * SPC
ACC_ROWS = SPC * SEG
OUT_PER_W = ACC_ROWS // NS

_mesh = plsc.VectorSubcoreMesh(
    core_axis_name="c", subcore_axis_name="s", num_cores=NC, num_subcores=NS
)

_cp = pltpu.CompilerParams()
if "needs_layout_passes" in pltpu.CompilerParams.__dataclass_fields__:
    _cp = dataclasses.replace(_cp, needs_layout_passes=False)


@functools.partial(
    pl.kernel,
    out_type=jax.ShapeDtypeStruct((STRIPS * BS, W), jnp.float32),
    mesh=_mesh,
    scratch_types=[
        pltpu.VMEM((ROWS_PER_W,), jnp.int32),
        pltpu.VMEM((ROWS_PER_W,), jnp.int32),
        pltpu.VMEM((ROWS_PER_W,), jnp.int32),
        pltpu.VMEM((H,), jnp.float32),
        pltpu.VMEM((H,), jnp.float32),
        [pltpu.VMEM((A_CHUNK, H), jnp.float32) for _ in range(2)],
        [pltpu.VMEM((A_CHUNK, H), jnp.float32) for _ in range(2)],
        [pltpu.SemaphoreType.DMA for _ in range(2)],
        [pltpu.SemaphoreType.DMA for _ in range(2)],
    ],
    compiler_params=_cp,
)
def _phase_a(ids_hbm, pos_hbm, tt_hbm, wtab, ptab, ttab, flat_out,
             widx, pidx, tidx, t0v, dtv, wrows, prows, gsems, wsems):
    wid = lax.axis_index("s") * NC + lax.axis_index("c")
    base = wid * ROWS_PER_W

    pltpu.sync_copy(ids_hbm.at[pl.ds(base, ROWS_PER_W)], widx)
    pltpu.sync_copy(pos_hbm.at[pl.ds(base, ROWS_PER_W)], pidx)
    pltpu.sync_copy(tt_hbm.at[pl.ds(base, ROWS_PER_W)], tidx)
    pltpu.sync_copy(ttab.at[0], t0v)
    pltpu.sync_copy(ttab.at[1], dtv)

    @pl.loop(0, H // L)
    def _(c):
        sl = pl.ds(c * L, L)
        dtv.at[sl][...] = dtv.at[sl][...] - t0v.at[sl][...]

    def issue_gathers(st, e):
        sl = pl.ds(st * A_CHUNK, A_CHUNK)
        pltpu.async_copy(wtab.at[widx.at[sl]], wrows[e], gsems[e])
        pltpu.async_copy(ptab.at[pidx.at[sl]], prows[e], gsems[e])

    def wait_gathers(st, e):
        sl = pl.ds(st * A_CHUNK, A_CHUNK)
        pltpu.make_async_copy(wtab.at[widx.at[sl]], wrows[e], gsems[e]).wait()
        pltpu.make_async_copy(ptab.at[pidx.at[sl]], prows[e], gsems[e]).wait()

    def compute(st, e):
        @pl.loop(0, A_CHUNK)
        def _(r):
            tt = plsc.load_gather(
                tidx, [jnp.full((L,), st * A_CHUNK + r, jnp.int32)])
            ttf = tt.astype(jnp.float32)
            for c in range(H // L):
                sl = (r, pl.ds(c * L, L))
                csl = pl.ds(c * L, L)
                wrows[e].at[*sl][...] = (
                    wrows[e].at[*sl][...] + prows[e].at[*sl][...]
                    + t0v.at[csl][...] + ttf * dtv.at[csl][...]
                )

    def _write_descs(st, e):
        off = base + st * A_CHUNK
        for k in range(STRIPS):
            yield pltpu.make_async_copy(
                wrows[e].at[pl.ds(0, A_CHUNK), pl.ds(k * W, W)],
                flat_out.at[pl.ds(k * BS + off, A_CHUNK)], wsems[e])

    def issue_writes(st, e):
        for d in _write_descs(st, e):
            d.start()

    def wait_writes(st, e):
        for d in _write_descs(st, e):
            d.wait()

    issue_gathers(0, 0)

    @pl.loop(0, A_STEPS // 2)
    def _(p):
        st0 = p * 2

        @pl.when(p > 0)
        def _():
            wait_writes(st0 - 1, 1)

        issue_gathers(st0 + 1, 1)
        wait_gathers(st0, 0)
        compute(st0, 0)
        issue_writes(st0, 0)

        @pl.when(p + 1 < A_STEPS // 2)
        def _():
            wait_writes(st0, 0)
            issue_gathers(st0 + 2, 0)

        wait_gathers(st0 + 1, 1)
        compute(st0 + 1, 1)
        issue_writes(st0 + 1, 1)

    wait_writes(A_STEPS - 2, 0)
    wait_writes(A_STEPS - 1, 1)


@functools.partial(
    pl.kernel,
    out_type=jax.ShapeDtypeStruct((NC * ACC_ROWS, W), jnp.float32),
    mesh=_mesh,
    scratch_types=[
        pltpu.VMEM((NNZ_PER_W,), jnp.int32),
        pltpu.VMEM((NNZ_PER_W,), jnp.int32),
        pltpu.VMEM((NNZ_PER_W,), jnp.int32),
        pltpu.VMEM((NNZ_PER_W,), jnp.float32),
        [pltpu.VMEM((B_CHUNK,), jnp.int32) for _ in range(2)],
        [pltpu.VMEM((B_CHUNK,), jnp.int32) for _ in range(2)],
        [pltpu.VMEM((B_CHUNK, W), jnp.float32) for _ in range(2)],
        pltpu.VMEM_SHARED((ACC_ROWS, W), jnp.float32),
        [pltpu.SemaphoreType.DMA for _ in range(2)],
        pltpu.SemaphoreType.DMA,
    ],
    compiler_params=_cp,
)
def _phase_b(flat_hbm, bidx_hbm, nidx_hbm, sidx_hbm, vals_hbm, out_hbm,
             tmpb, tmpn, tmps, vals_v, segs, srcs, rows, acc, gsems, zsem):
    cid = lax.axis_index("c")
    sid = lax.axis_index("s")
    nbase = sid * NNZ_PER_W

    pltpu.sync_copy(bidx_hbm.at[pl.ds(nbase, NNZ_PER_W)], tmpb)
    pltpu.sync_copy(nidx_hbm.at[pl.ds(nbase, NNZ_PER_W)], tmpn)
    pltpu.sync_copy(sidx_hbm.at[pl.ds(nbase, NNZ_PER_W)], tmps)
    pltpu.sync_copy(vals_hbm.at[pl.ds(nbase, NNZ_PER_W)], vals_v)

    @pl.loop(0, B_CHUNK)
    def _(r):
        for c in range(W // L):
            rows[0].at[r, pl.ds(c * L, L)][...] = jnp.zeros((L,), jnp.float32)

    @pl.loop(0, OUT_PER_W // B_CHUNK)
    def _(z):
        pltpu.async_copy(
            rows[0], acc.at[pl.ds(sid * OUT_PER_W + z * B_CHUNK, B_CHUNK)], zsem)

    @pl.loop(0, OUT_PER_W // B_CHUNK)
    def _(z):
        pltpu.make_async_copy(
            rows[0], acc.at[pl.ds(sid * OUT_PER_W + z * B_CHUNK, B_CHUNK)],
            zsem).wait()

    plsc.subcore_barrier()

    def compute_idx(i, e):
        st = i // SPC
        k = i % SPC
        for c in range(B_CHUNK // L):
            sl = pl.ds(st * B_CHUNK + c * L, L)
            dst = pl.ds(c * L, L)
            bb = tmpb.at[sl][...]
            segs[e].at[dst][...] = bb * N + tmpn.at[sl][...] + k * SEG
            srcs[e].at[dst][...] = (
                bb * S + tmps.at[sl][...] + (cid * SPC + k) * BS)

    def issue_gather(e):
        pltpu.async_copy(flat_hbm.at[srcs[e]], rows[e], gsems[e])

    def wait_gather(e):
        pltpu.make_async_copy(flat_hbm.at[srcs[e]], rows[e], gsems[e]).wait()

    def scale(i, e):
        st = i // SPC

        @pl.loop(0, B_CHUNK)
        def _(r):
            vb = plsc.load_gather(
                vals_v, [jnp.full((L,), st * B_CHUNK + r, jnp.int32)])
            for c in range(W // L):
                sl = (r, pl.ds(c * L, L))
                rows[e].at[*sl][...] = rows[e].at[*sl][...] * vb

    def scatter(e):
        pltpu.sync_copy(rows[e], acc.at[segs[e]], add=True)

    compute_idx(0, 0)
    issue_gather(0)

    @pl.loop(0, B_ITEMS // 2)
    def _(p):
        i0 = p * 2
        compute_idx(i0 + 1, 1)
        issue_gather(1)
        wait_gather(0)
        scale(i0, 0)
        scatter(0)

        @pl.when(p + 1 < B_ITEMS // 2)
        def _():
            compute_idx(i0 + 2, 0)
            issue_gather(0)

        wait_gather(1)
        scale(i0 + 1, 1)
        scatter(1)

    plsc.subcore_barrier()

    @pl.loop(0, OUT_PER_W // B_CHUNK)
    def _(z):
        ro = sid * OUT_PER_W + z * B_CHUNK
        pltpu.async_copy(acc.at[pl.ds(ro, B_CHUNK)],
                         out_hbm.at[pl.ds(cid * ACC_ROWS + ro, B_CHUNK)], zsem)

    @pl.loop(0, OUT_PER_W // B_CHUNK)
    def _(z):
        ro = sid * OUT_PER_W + z * B_CHUNK
        pltpu.make_async_copy(
            acc.at[pl.ds(ro, B_CHUNK)],
            out_hbm.at[pl.ds(cid * ACC_ROWS + ro, B_CHUNK)], zsem).wait()


_LN_R = 256


def _ln_body(x_ref, g_ref, b_ref, o_ref):
    x = jnp.concatenate([x_ref[k] for k in range(STRIPS)], axis=-1)
    mu = jnp.mean(x, axis=-1, keepdims=True)
    xc = x - mu
    var = jnp.mean(xc * xc, axis=-1, keepdims=True)
    o_ref[...] = xc * lax.rsqrt(var + EPS) * g_ref[...] + b_ref[...]


def kernel(input_ids, mask_indices, mask_values, position_ids, token_type_ids,
           word_emb, pos_emb, type_emb, ln_gamma, ln_beta):
    ids = jnp.asarray(input_ids, jnp.int32).reshape(BS)
    pos = jnp.asarray(position_ids, jnp.int32).reshape(BS)
    tts = jnp.asarray(token_type_ids, jnp.int32).reshape(BS)
    mask = jnp.asarray(mask_indices, jnp.int32)

    flat = _phase_a(ids, pos, tts, word_emb, pos_emb, type_emb)
    node6 = _phase_b(flat, mask[0], mask[1], mask[2], mask_values)

    out = pl.pallas_call(
        _ln_body,
        grid=(SEG // _LN_R,),
        in_specs=[
            pl.BlockSpec((STRIPS, _LN_R, W), lambda i: (0, i, 0)),
            pl.BlockSpec((1, H), lambda i: (0, 0)),
            pl.BlockSpec((1, H), lambda i: (0, 0)),
        ],
        out_specs=pl.BlockSpec((_LN_R, H), lambda i: (i, 0)),
        out_shape=jax.ShapeDtypeStruct((SEG, H), jnp.float32),
    )(node6.reshape(STRIPS, SEG, W), ln_gamma.reshape(1, H), ln_beta.reshape(1, H))

    return out.reshape(B, N, H)


# speedup vs baseline: 2.5722x; 1.0122x over previous
import dataclasses
import functools

import jax
import jax.numpy as jnp
from jax import lax
from jax.experimental import pallas as pl
from jax.experimental.pallas import tpu as pltpu
from jax.experimental.pallas import tpu_sc as plsc

NC, NS, L = 2, 16, 16
NW = NC * NS
B = 16
S = 512
N = 256
H = 768
W = 128
STRIPS = H // W
SPC = STRIPS // NC
BS = B * S
SEG = B * N
NNZ = 16384
EPS = 1e-12

ROWS_PER_W = BS // NW
A_CHUNK = 32
A_STEPS = ROWS_PER_W // A_CHUNK

NNZ_PER_W = NNZ // NS
B_CHUNK = 64
B_STEPS = NNZ_PER_W // B_CHUNK
B_ITEMS = B_STEPS * SPC
ACC_ROWS = SPC * SEG
OUT_PER_W = ACC_ROWS // NS

_mesh = plsc.VectorSubcoreMesh(
    core_axis_name="c", subcore_axis_name="s", num_cores=NC, num_subcores=NS
)

_cp = pltpu.CompilerParams()
if "needs_layout_passes" in pltpu.CompilerParams.__dataclass_fields__:
    _cp = dataclasses.replace(_cp, needs_layout_passes=False)


@functools.partial(
    pl.kernel,
    out_type=jax.ShapeDtypeStruct((STRIPS * BS, W), jnp.float32),
    mesh=_mesh,
    scratch_types=[
        pltpu.VMEM((ROWS_PER_W,), jnp.int32),
        pltpu.VMEM((ROWS_PER_W,), jnp.int32),
        pltpu.VMEM((ROWS_PER_W,), jnp.int32),
        pltpu.VMEM((H,), jnp.float32),
        pltpu.VMEM((H,), jnp.float32),
        [pltpu.VMEM((A_CHUNK, H), jnp.float32) for _ in range(2)],
        [pltpu.VMEM((A_CHUNK, H), jnp.float32) for _ in range(2)],
        [pltpu.SemaphoreType.DMA for _ in range(2)],
        [pltpu.SemaphoreType.DMA for _ in range(2)],
    ],
    compiler_params=_cp,
)
def _phase_a(ids_hbm, pos_hbm, tt_hbm, wtab, ptab, ttab, flat_out,
             widx, pidx, tidx, t0v, dtv, wrows, prows, gsems, wsems):
    wid = lax.axis_index("s") * NC + lax.axis_index("c")
    base = wid * ROWS_PER_W

    pltpu.sync_copy(ids_hbm.at[pl.ds(base, ROWS_PER_W)], widx)
    pltpu.sync_copy(pos_hbm.at[pl.ds(base, ROWS_PER_W)], pidx)
    pltpu.sync_copy(tt_hbm.at[pl.ds(base, ROWS_PER_W)], tidx)
    pltpu.sync_copy(ttab.at[0], t0v)
    pltpu.sync_copy(ttab.at[1], dtv)

    @pl.loop(0, H // L)
    def _(c):
        sl = pl.ds(c * L, L)
        dtv.at[sl][...] = dtv.at[sl][...] - t0v.at[sl][...]

    def issue_gathers(st, e):
        sl = pl.ds(st * A_CHUNK, A_CHUNK)
        pltpu.async_copy(wtab.at[widx.at[sl]], wrows[e], gsems[e])
        pltpu.async_copy(ptab.at[pidx.at[sl]], prows[e], gsems[e])

    def wait_gathers(st, e):
        sl = pl.ds(st * A_CHUNK, A_CHUNK)
        pltpu.make_async_copy(wtab.at[widx.at[sl]], wrows[e], gsems[e]).wait()
        pltpu.make_async_copy(ptab.at[pidx.at[sl]], prows[e], gsems[e]).wait()

    def compute(st, e):
        @pl.loop(0, A_CHUNK)
        def _(r):
            tt = plsc.load_gather(
                tidx, [jnp.full((L,), st * A_CHUNK + r, jnp.int32)])
            ttf = tt.astype(jnp.float32)
            for c in range(H // L):
                sl = (r, pl.ds(c * L, L))
                csl = pl.ds(c * L, L)
                wrows[e].at[*sl][...] = (
                    wrows[e].at[*sl][...] + prows[e].at[*sl][...]
                    + t0v.at[csl][...] + ttf * dtv.at[csl][...]
                )

    def _write_descs(st, e):
        off = base + st * A_CHUNK
        for k in range(STRIPS):
            yield pltpu.make_async_copy(
                wrows[e].at[pl.ds(0, A_CHUNK), pl.ds(k * W, W)],
                flat_out.at[pl.ds(k * BS + off, A_CHUNK)], wsems[e])

    def issue_writes(st, e):
        for d in _write_descs(st, e):
            d.start()

    def wait_writes(st, e):
        for d in _write_descs(st, e):
            d.wait()

    issue_gathers(0, 0)

    @pl.loop(0, A_STEPS // 2)
    def _(p):
        st0 = p * 2

        @pl.when(p > 0)
        def _():
            wait_writes(st0 - 1, 1)

        issue_gathers(st0 + 1, 1)
        wait_gathers(st0, 0)
        compute(st0, 0)
        issue_writes(st0, 0)

        @pl.when(p + 1 < A_STEPS // 2)
        def _():
            wait_writes(st0, 0)
            issue_gathers(st0 + 2, 0)

        wait_gathers(st0 + 1, 1)
        compute(st0 + 1, 1)
        issue_writes(st0 + 1, 1)

    wait_writes(A_STEPS - 2, 0)
    wait_writes(A_STEPS - 1, 1)


@functools.partial(
    pl.kernel,
    out_type=jax.ShapeDtypeStruct((NC * ACC_ROWS, W), jnp.float32),
    mesh=_mesh,
    scratch_types=[
        pltpu.VMEM((NNZ_PER_W,), jnp.int32),
        pltpu.VMEM((NNZ_PER_W,), jnp.int32),
        pltpu.VMEM((NNZ_PER_W,), jnp.int32),
        pltpu.VMEM((NNZ_PER_W,), jnp.float32),
        [pltpu.VMEM((B_CHUNK,), jnp.int32) for _ in range(2)],
        [pltpu.VMEM((B_CHUNK,), jnp.int32) for _ in range(2)],
        [pltpu.VMEM((B_CHUNK, W), jnp.float32) for _ in range(2)],
        pltpu.VMEM_SHARED((ACC_ROWS, W), jnp.float32),
        [pltpu.SemaphoreType.DMA for _ in range(2)],
        [pltpu.SemaphoreType.DMA for _ in range(2)],
        pltpu.SemaphoreType.DMA,
    ],
    compiler_params=_cp,
)
def _phase_b(flat_hbm, bidx_hbm, nidx_hbm, sidx_hbm, vals_hbm, out_hbm,
             tmpb, tmpn, tmps, vals_v, segs, srcs, rows, acc, gsems, ssems, zsem):
    cid = lax.axis_index("c")
    sid = lax.axis_index("s")
    nbase = sid * NNZ_PER_W

    _stage = [
        pltpu.make_async_copy(bidx_hbm.at[pl.ds(nbase, NNZ_PER_W)], tmpb, zsem),
        pltpu.make_async_copy(nidx_hbm.at[pl.ds(nbase, NNZ_PER_W)], tmpn, zsem),
        pltpu.make_async_copy(sidx_hbm.at[pl.ds(nbase, NNZ_PER_W)], tmps, zsem),
        pltpu.make_async_copy(vals_hbm.at[pl.ds(nbase, NNZ_PER_W)], vals_v, zsem),
    ]
    for d in _stage:
        d.start()
    for d in _stage:
        d.wait()

    @pl.loop(0, B_CHUNK)
    def _(r):
        for c in range(W // L):
            rows[0].at[r, pl.ds(c * L, L)][...] = jnp.zeros((L,), jnp.float32)

    @pl.loop(0, OUT_PER_W // B_CHUNK)
    def _(z):
        pltpu.async_copy(
            rows[0], acc.at[pl.ds(sid * OUT_PER_W + z * B_CHUNK, B_CHUNK)], zsem)

    @pl.loop(0, OUT_PER_W // B_CHUNK)
    def _(z):
        pltpu.make_async_copy(
            rows[0], acc.at[pl.ds(sid * OUT_PER_W + z * B_CHUNK, B_CHUNK)],
            zsem).wait()

    plsc.subcore_barrier()

    def compute_idx(i, e):
        st = i // SPC
        k = i % SPC
        for c in range(B_CHUNK // L):
            sl = pl.ds(st * B_CHUNK + c * L, L)
            dst = pl.ds(c * L, L)
            bb = tmpb.at[sl][...]
            segs[e].at[dst][...] = bb * N + tmpn.at[sl][...] + k * SEG
            srcs[e].at[dst][...] = (
                bb * S + tmps.at[sl][...] + (cid * SPC + k) * BS)

    def issue_gather(e):
        pltpu.async_copy(flat_hbm.at[srcs[e]], rows[e], gsems[e])

    def wait_gather(e):
        pltpu.make_async_copy(flat_hbm.at[srcs[e]], rows[e], gsems[e]).wait()

    def scale(i, e):
        st = i // SPC

        @pl.loop(0, B_CHUNK)
        def _(r):
            vb = plsc.load_gather(
                vals_v, [jnp.full((L,), st * B_CHUNK + r, jnp.int32)])
            for c in range(W // L):
                sl = (r, pl.ds(c * L, L))
                rows[e].at[*sl][...] = rows[e].at[*sl][...] * vb

    def issue_scatter(e):
        pltpu.async_copy(rows[e], acc.at[segs[e]], ssems[e], add=True)

    def wait_scatter(e):
        pltpu.make_async_copy(rows[e], acc.at[segs[e]], ssems[e]).wait()

    compute_idx(0, 0)
    issue_gather(0)

    @pl.loop(0, B_ITEMS // 2)
    def _(p):
        i0 = p * 2

        @pl.when(p > 0)
        def _():
            wait_scatter(1)

        compute_idx(i0 + 1, 1)
        issue_gather(1)
        wait_gather(0)
        scale(i0, 0)
        issue_scatter(0)

        @pl.when(p + 1 < B_ITEMS // 2)
        def _():
            wait_scatter(0)
            compute_idx(i0 + 2, 0)
            issue_gather(0)

        wait_gather(1)
        scale(i0 + 1, 1)
        issue_scatter(1)

    wait_scatter(0)
    wait_scatter(1)

    plsc.subcore_barrier()

    @pl.loop(0, OUT_PER_W // B_CHUNK)
    def _(z):
        ro = sid * OUT_PER_W + z * B_CHUNK
        pltpu.async_copy(acc.at[pl.ds(ro, B_CHUNK)],
                         out_hbm.at[pl.ds(cid * ACC_ROWS + ro, B_CHUNK)], zsem)

    @pl.loop(0, OUT_PER_W // B_CHUNK)
    def _(z):
        ro = sid * OUT_PER_W + z * B_CHUNK
        pltpu.make_async_copy(
            acc.at[pl.ds(ro, B_CHUNK)],
            out_hbm.at[pl.ds(cid * ACC_ROWS + ro, B_CHUNK)], zsem).wait()


_LN_R = 256


def _ln_body(x_ref, g_ref, b_ref, o_ref):
    x = jnp.concatenate([x_ref[k] for k in range(STRIPS)], axis=-1)
    mu = jnp.mean(x, axis=-1, keepdims=True)
    xc = x - mu
    var = jnp.mean(xc * xc, axis=-1, keepdims=True)
    o_ref[...] = xc * lax.rsqrt(var + EPS) * g_ref[...] + b_ref[...]


def kernel(input_ids, mask_indices, mask_values, position_ids, token_type_ids,
           word_emb, pos_emb, type_emb, ln_gamma, ln_beta):
    ids = jnp.asarray(input_ids, jnp.int32).reshape(BS)
    pos = jnp.asarray(position_ids, jnp.int32).reshape(BS)
    tts = jnp.asarray(token_type_ids, jnp.int32).reshape(BS)
    mask = jnp.asarray(mask_indices, jnp.int32)

    flat = _phase_a(ids, pos, tts, word_emb, pos_emb, type_emb)
    node6 = _phase_b(flat, mask[0], mask[1], mask[2], mask_values)

    out = pl.pallas_call(
        _ln_body,
        grid=(SEG // _LN_R,),
        in_specs=[
            pl.BlockSpec((STRIPS, _LN_R, W), lambda i: (0, i, 0)),
            pl.BlockSpec((1, H), lambda i: (0, 0)),
            pl.BlockSpec((1, H), lambda i: (0, 0)),
        ],
        out_specs=pl.BlockSpec((_LN_R, H), lambda i: (i, 0)),
        out_shape=jax.ShapeDtypeStruct((SEG, H), jnp.float32),
    )(node6.reshape(STRIPS, SEG, W), ln_gamma.reshape(1, H), ln_beta.reshape(1, H))

    return out.reshape(B, N, H)


# speedup vs baseline: 2.6226x; 1.0196x over previous
import dataclasses
import functools

import jax
import jax.numpy as jnp
from jax import lax
from jax.experimental import pallas as pl
from jax.experimental.pallas import tpu as pltpu
from jax.experimental.pallas import tpu_sc as plsc

NC, NS, L = 2, 16, 16
NW = NC * NS
B = 16
S = 512
N = 256
H = 768
W = 128
STRIPS = H // W
SPC = STRIPS // NC
BS = B * S
SEG = B * N
NNZ = 16384
EPS = 1e-12

ROWS_PER_W = BS // NW
A_CHUNK = 32
A_STEPS = ROWS_PER_W // A_CHUNK

NNZ_PER_W = NNZ // NS
B_CHUNK = 64
B_STEPS = NNZ_PER_W // B_CHUNK
B_ITEMS = B_STEPS * SPC
ACC_ROWS = SPC * SEG
OUT_PER_W = ACC_ROWS // NS

_mesh = plsc.VectorSubcoreMesh(
    core_axis_name="c", subcore_axis_name="s", num_cores=NC, num_subcores=NS
)

_cp = pltpu.CompilerParams()
if "needs_layout_passes" in pltpu.CompilerParams.__dataclass_fields__:
    _cp = dataclasses.replace(_cp, needs_layout_passes=False)


@functools.partial(
    pl.kernel,
    out_type=jax.ShapeDtypeStruct((STRIPS * BS, W), jnp.float32),
    mesh=_mesh,
    scratch_types=[
        pltpu.VMEM((ROWS_PER_W,), jnp.int32),
        pltpu.VMEM((ROWS_PER_W,), jnp.int32),
        pltpu.VMEM((ROWS_PER_W,), jnp.int32),
        pltpu.VMEM((H,), jnp.float32),
        pltpu.VMEM((H,), jnp.float32),
        [pltpu.VMEM((A_CHUNK, H), jnp.float32) for _ in range(2)],
        [pltpu.VMEM((A_CHUNK, H), jnp.float32) for _ in range(2)],
        [pltpu.SemaphoreType.DMA for _ in range(2)],
        [pltpu.SemaphoreType.DMA for _ in range(2)],
    ],
    compiler_params=_cp,
)
def _phase_a(ids_hbm, pos_hbm, tt_hbm, wtab, ptab, ttab, flat_out,
             widx, pidx, tidx, t0v, dtv, wrows, prows, gsems, wsems):
    wid = lax.axis_index("s") * NC + lax.axis_index("c")
    base = wid * ROWS_PER_W

    _stage = [
        pltpu.make_async_copy(ids_hbm.at[pl.ds(base, ROWS_PER_W)], widx, wsems[0]),
        pltpu.make_async_copy(pos_hbm.at[pl.ds(base, ROWS_PER_W)], pidx, wsems[0]),
        pltpu.make_async_copy(tt_hbm.at[pl.ds(base, ROWS_PER_W)], tidx, wsems[0]),
        pltpu.make_async_copy(ttab.at[0], t0v, wsems[0]),
        pltpu.make_async_copy(ttab.at[1], dtv, wsems[0]),
    ]
    for d in _stage:
        d.start()
    for d in _stage:
        d.wait()

    @pl.loop(0, H // L)
    def _(c):
        sl = pl.ds(c * L, L)
        dtv.at[sl][...] = dtv.at[sl][...] - t0v.at[sl][...]

    def issue_gathers(st, e):
        sl = pl.ds(st * A_CHUNK, A_CHUNK)
        pltpu.async_copy(wtab.at[widx.at[sl]], wrows[e], gsems[e])
        pltpu.async_copy(ptab.at[pidx.at[sl]], prows[e], gsems[e])

    def wait_gathers(st, e):
        sl = pl.ds(st * A_CHUNK, A_CHUNK)
        pltpu.make_async_copy(wtab.at[widx.at[sl]], wrows[e], gsems[e]).wait()
        pltpu.make_async_copy(ptab.at[pidx.at[sl]], prows[e], gsems[e]).wait()

    def compute(st, e):
        @pl.loop(0, A_CHUNK)
        def _(r):
            tt = plsc.load_gather(
                tidx, [jnp.full((L,), st * A_CHUNK + r, jnp.int32)])
            ttf = tt.astype(jnp.float32)
            for c in range(H // L):
                sl = (r, pl.ds(c * L, L))
                csl = pl.ds(c * L, L)
                wrows[e].at[*sl][...] = (
                    wrows[e].at[*sl][...] + prows[e].at[*sl][...]
                    + t0v.at[csl][...] + ttf * dtv.at[csl][...]
                )

    def _write_descs(st, e):
        off = base + st * A_CHUNK
        for k in range(STRIPS):
            yield pltpu.make_async_copy(
                wrows[e].at[pl.ds(0, A_CHUNK), pl.ds(k * W, W)],
                flat_out.at[pl.ds(k * BS + off, A_CHUNK)], wsems[e])

    def issue_writes(st, e):
        for d in _write_descs(st, e):
            d.start()

    def wait_writes(st, e):
        for d in _write_descs(st, e):
            d.wait()

    issue_gathers(0, 0)

    @pl.loop(0, A_STEPS // 2)
    def _(p):
        st0 = p * 2

        @pl.when(p > 0)
        def _():
            wait_writes(st0 - 1, 1)

        issue_gathers(st0 + 1, 1)
        wait_gathers(st0, 0)
        compute(st0, 0)
        issue_writes(st0, 0)

        @pl.when(p + 1 < A_STEPS // 2)
        def _():
            wait_writes(st0, 0)
            issue_gathers(st0 + 2, 0)

        wait_gathers(st0 + 1, 1)
        compute(st0 + 1, 1)
        issue_writes(st0 + 1, 1)

    wait_writes(A_STEPS - 2, 0)
    wait_writes(A_STEPS - 1, 1)


@functools.partial(
    pl.kernel,
    out_type=jax.ShapeDtypeStruct((NC * ACC_ROWS, W), jnp.float32),
    mesh=_mesh,
    scratch_types=[
        pltpu.VMEM((NNZ_PER_W,), jnp.int32),
        pltpu.VMEM((NNZ_PER_W,), jnp.int32),
        pltpu.VMEM((NNZ_PER_W,), jnp.int32),
        pltpu.VMEM((NNZ_PER_W,), jnp.float32),
        [pltpu.VMEM((B_CHUNK,), jnp.int32) for _ in range(2)],
        [pltpu.VMEM((B_CHUNK,), jnp.int32) for _ in range(2)],
        [pltpu.VMEM((B_CHUNK, W), jnp.float32) for _ in range(2)],
        pltpu.VMEM_SHARED((ACC_ROWS, W), jnp.float32),
        [pltpu.SemaphoreType.DMA for _ in range(2)],
        [pltpu.SemaphoreType.DMA for _ in range(2)],
        pltpu.SemaphoreType.DMA,
    ],
    compiler_params=_cp,
)
def _phase_b(flat_hbm, bidx_hbm, nidx_hbm, sidx_hbm, vals_hbm, out_hbm,
             tmpb, tmpn, tmps, vals_v, segs, srcs, rows, acc, gsems, ssems, zsem):
    cid = lax.axis_index("c")
    sid = lax.axis_index("s")
    nbase = sid * NNZ_PER_W

    _stage = [
        pltpu.make_async_copy(bidx_hbm.at[pl.ds(nbase, NNZ_PER_W)], tmpb, zsem),
        pltpu.make_async_copy(nidx_hbm.at[pl.ds(nbase, NNZ_PER_W)], tmpn, zsem),
        pltpu.make_async_copy(sidx_hbm.at[pl.ds(nbase, NNZ_PER_W)], tmps, zsem),
        pltpu.make_async_copy(vals_hbm.at[pl.ds(nbase, NNZ_PER_W)], vals_v, zsem),
    ]
    for d in _stage:
        d.start()
    for d in _stage:
        d.wait()

    @pl.loop(0, B_CHUNK)
    def _(r):
        for c in range(W // L):
            rows[0].at[r, pl.ds(c * L, L)][...] = jnp.zeros((L,), jnp.float32)

    @pl.loop(0, OUT_PER_W // B_CHUNK)
    def _(z):
        pltpu.async_copy(
            rows[0], acc.at[pl.ds(sid * OUT_PER_W + z * B_CHUNK, B_CHUNK)], zsem)

    @pl.loop(0, OUT_PER_W // B_CHUNK)
    def _(z):
        pltpu.make_async_copy(
            rows[0], acc.at[pl.ds(sid * OUT_PER_W + z * B_CHUNK, B_CHUNK)],
            zsem).wait()

    plsc.subcore_barrier()

    def compute_idx(i, e):
        st = i // SPC
        k = i % SPC
        for c in range(B_CHUNK // L):
            sl = pl.ds(st * B_CHUNK + c * L, L)
            dst = pl.ds(c * L, L)
            bb = tmpb.at[sl][...]
            segs[e].at[dst][...] = bb * N + tmpn.at[sl][...] + k * SEG
            srcs[e].at[dst][...] = (
                bb * S + tmps.at[sl][...] + (cid * SPC + k) * BS)

    def issue_gather(e):
        pltpu.async_copy(flat_hbm.at[srcs[e]], rows[e], gsems[e])

    def wait_gather(e):
        pltpu.make_async_copy(flat_hbm.at[srcs[e]], rows[e], gsems[e]).wait()

    def scale(i, e):
        st = i // SPC

        @pl.loop(0, B_CHUNK)
        def _(r):
            vb = plsc.load_gather(
                vals_v, [jnp.full((L,), st * B_CHUNK + r, jnp.int32)])
            for c in range(W // L):
                sl = (r, pl.ds(c * L, L))
                rows[e].at[*sl][...] = rows[e].at[*sl][...] * vb

    def issue_scatter(e):
        pltpu.async_copy(rows[e], acc.at[segs[e]], ssems[e], add=True)

    def wait_scatter(e):
        pltpu.make_async_copy(rows[e], acc.at[segs[e]], ssems[e]).wait()

    compute_idx(0, 0)
    issue_gather(0)

    @pl.loop(0, B_ITEMS // 2)
    def _(p):
        i0 = p * 2

        @pl.when(p > 0)
        def _():
            wait_scatter(1)

        compute_idx(i0 + 1, 1)
        issue_gather(1)
        wait_gather(0)
        scale(i0, 0)
        issue_scatter(0)

        @pl.when(p + 1 < B_ITEMS // 2)
        def _():
            wait_scatter(0)
            compute_idx(i0 + 2, 0)
            issue_gather(0)

        wait_gather(1)
        scale(i0 + 1, 1)
        issue_scatter(1)

    wait_scatter(0)
    wait_scatter(1)

    plsc.subcore_barrier()

    @pl.loop(0, OUT_PER_W // B_CHUNK)
    def _(z):
        ro = sid * OUT_PER_W + z * B_CHUNK
        pltpu.async_copy(acc.at[pl.ds(ro, B_CHUNK)],
                         out_hbm.at[pl.ds(cid * ACC_ROWS + ro, B_CHUNK)], zsem)

    @pl.loop(0, OUT_PER_W // B_CHUNK)
    def _(z):
        ro = sid * OUT_PER_W + z * B_CHUNK
        pltpu.make_async_copy(
            acc.at[pl.ds(ro, B_CHUNK)],
            out_hbm.at[pl.ds(cid * ACC_ROWS + ro, B_CHUNK)], zsem).wait()


_LN_R = 512


def _ln_body(x_ref, g_ref, b_ref, o_ref):
    xs = [x_ref[k] for k in range(STRIPS)]
    s1 = sum(jnp.sum(x, axis=-1, keepdims=True) for x in xs)
    s2 = sum(jnp.sum(x * x, axis=-1, keepdims=True) for x in xs)
    mu = s1 * (1.0 / H)
    var = s2 * (1.0 / H) - mu * mu
    inv = lax.rsqrt(var + EPS)
    for k in range(STRIPS):
        o_ref[:, k * W:(k + 1) * W] = (
            (xs[k] - mu) * inv * g_ref[0, k * W:(k + 1) * W]
            + b_ref[0, k * W:(k + 1) * W])


def kernel(input_ids, mask_indices, mask_values, position_ids, token_type_ids,
           word_emb, pos_emb, type_emb, ln_gamma, ln_beta):
    ids = jnp.asarray(input_ids, jnp.int32).reshape(BS)
    pos = jnp.asarray(position_ids, jnp.int32).reshape(BS)
    tts = jnp.asarray(token_type_ids, jnp.int32).reshape(BS)
    mask = jnp.asarray(mask_indices, jnp.int32)

    flat = _phase_a(ids, pos, tts, word_emb, pos_emb, type_emb)
    node6 = _phase_b(flat, mask[0], mask[1], mask[2], mask_values)

    out = pl.pallas_call(
        _ln_body,
        grid=(SEG // _LN_R,),
        in_specs=[
            pl.BlockSpec((STRIPS, _LN_R, W), lambda i: (0, i, 0)),
            pl.BlockSpec((1, H), lambda i: (0, 0)),
            pl.BlockSpec((1, H), lambda i: (0, 0)),
        ],
        out_specs=pl.BlockSpec((_LN_R, H), lambda i: (i, 0)),
        out_shape=jax.ShapeDtypeStruct((SEG, H), jnp.float32),
    )(node6.reshape(STRIPS, SEG, W), ln_gamma.reshape(1, H), ln_beta.reshape(1, H))

    return out.reshape(B, N, H)


# speedup vs baseline: 3.7041x; 1.4124x over previous
import dataclasses
import functools

import jax
import jax.numpy as jnp
from jax import lax
from jax.experimental import pallas as pl
from jax.experimental.pallas import tpu as pltpu
from jax.experimental.pallas import tpu_sc as plsc

NC, NS, L = 2, 16, 16
NW = NC * NS
B = 16
S = 512
N = 256
H = 768
MAXPOS = 512
TYPES = 2
W = 128
STRIPS = H // W
SPC = STRIPS // NC
BS = B * S
SEG = B * N
NNZ = 16384
EPS = 1e-12

ROWS_PER_W = BS // NW
A_CHUNK = 32
A_STEPS = ROWS_PER_W // A_CHUNK

NNZ_PER_W = NNZ // NS
B_CHUNK = 64
B_STEPS = NNZ_PER_W // B_CHUNK
B_ITEMS = B_STEPS * SPC
ACC_ROWS = SPC * SEG
OUT_PER_W = ACC_ROWS // NS

_mesh = plsc.VectorSubcoreMesh(
    core_axis_name="c", subcore_axis_name="s", num_cores=NC, num_subcores=NS
)

_cp = pltpu.CompilerParams()
if "needs_layout_passes" in pltpu.CompilerParams.__dataclass_fields__:
    _cp = dataclasses.replace(_cp, needs_layout_passes=False)


@functools.partial(
    pl.kernel,
    out_type=jax.ShapeDtypeStruct((STRIPS * BS, W), jnp.float32),
    mesh=_mesh,
    scratch_types=[
        pltpu.VMEM((ROWS_PER_W,), jnp.int32),
        pltpu.VMEM((ROWS_PER_W,), jnp.int32),
        pltpu.VMEM((ROWS_PER_W,), jnp.int32),
        [pltpu.VMEM((A_CHUNK, H), jnp.float32) for _ in range(2)],
        [pltpu.VMEM((A_CHUNK, H), jnp.float32) for _ in range(2)],
        [pltpu.SemaphoreType.DMA for _ in range(2)],
        [pltpu.SemaphoreType.DMA for _ in range(2)],
    ],
    compiler_params=_cp,
)
def _phase_a(ids_hbm, pos_hbm, tt_hbm, wtab, pttab, flat_out,
             widx, pidx, tidx, wrows, prows, gsems, wsems):
    wid = lax.axis_index("s") * NC + lax.axis_index("c")
    base = wid * ROWS_PER_W

    _stage = [
        pltpu.make_async_copy(ids_hbm.at[pl.ds(base, ROWS_PER_W)], widx, wsems[0]),
        pltpu.make_async_copy(pos_hbm.at[pl.ds(base, ROWS_PER_W)], pidx, wsems[0]),
        pltpu.make_async_copy(tt_hbm.at[pl.ds(base, ROWS_PER_W)], tidx, wsems[0]),
    ]
    for d in _stage:
        d.start()
    for d in _stage:
        d.wait()

    @pl.loop(0, ROWS_PER_W // L)
    def _(c):
        sl = pl.ds(c * L, L)
        pidx.at[sl][...] = pidx.at[sl][...] + tidx.at[sl][...] * MAXPOS

    def issue_gathers(st, e):
        sl = pl.ds(st * A_CHUNK, A_CHUNK)
        pltpu.async_copy(wtab.at[widx.at[sl]], wrows[e], gsems[e])
        pltpu.async_copy(pttab.at[pidx.at[sl]], prows[e], gsems[e])

    def wait_gathers(st, e):
        sl = pl.ds(st * A_CHUNK, A_CHUNK)
        pltpu.make_async_copy(wtab.at[widx.at[sl]], wrows[e], gsems[e]).wait()
        pltpu.make_async_copy(pttab.at[pidx.at[sl]], prows[e], gsems[e]).wait()

    def compute(st, e):
        @pl.loop(0, A_CHUNK)
        def _(r):
            for c in range(H // L):
                sl = (r, pl.ds(c * L, L))
                wrows[e].at[*sl][...] = (
                    wrows[e].at[*sl][...] + prows[e].at[*sl][...])

    def _write_descs(st, e):
        off = base + st * A_CHUNK
        for k in range(STRIPS):
            yield pltpu.make_async_copy(
                wrows[e].at[pl.ds(0, A_CHUNK), pl.ds(k * W, W)],
                flat_out.at[pl.ds(k * BS + off, A_CHUNK)], wsems[e])

    def issue_writes(st, e):
        for d in _write_descs(st, e):
            d.start()

    def wait_writes(st, e):
        for d in _write_descs(st, e):
            d.wait()

    issue_gathers(0, 0)

    @pl.loop(0, A_STEPS // 2)
    def _(p):
        st0 = p * 2

        @pl.when(p > 0)
        def _():
            wait_writes(st0 - 1, 1)

        issue_gathers(st0 + 1, 1)
        wait_gathers(st0, 0)
        compute(st0, 0)
        issue_writes(st0, 0)

        @pl.when(p + 1 < A_STEPS // 2)
        def _():
            wait_writes(st0, 0)
            issue_gathers(st0 + 2, 0)

        wait_gathers(st0 + 1, 1)
        compute(st0 + 1, 1)
        issue_writes(st0 + 1, 1)

    wait_writes(A_STEPS - 2, 0)
    wait_writes(A_STEPS - 1, 1)


@functools.partial(
    pl.kernel,
    out_type=jax.ShapeDtypeStruct((NC * ACC_ROWS, W), jnp.float32),
    mesh=_mesh,
    scratch_types=[
        pltpu.VMEM((NNZ_PER_W,), jnp.int32),
        pltpu.VMEM((NNZ_PER_W,), jnp.int32),
        pltpu.VMEM((NNZ_PER_W,), jnp.int32),
        pltpu.VMEM((NNZ_PER_W,), jnp.float32),
        [pltpu.VMEM((B_CHUNK,), jnp.int32) for _ in range(2)],
        [pltpu.VMEM((B_CHUNK,), jnp.int32) for _ in range(2)],
        [pltpu.VMEM((B_CHUNK, W), jnp.float32) for _ in range(2)],
        pltpu.VMEM_SHARED((ACC_ROWS, W), jnp.float32),
        [pltpu.SemaphoreType.DMA for _ in range(2)],
        [pltpu.SemaphoreType.DMA for _ in range(2)],
        pltpu.SemaphoreType.DMA,
    ],
    compiler_params=_cp,
)
def _phase_b(flat_hbm, bidx_hbm, nidx_hbm, sidx_hbm, vals_hbm, out_hbm,
             tmpb, tmpn, tmps, vals_v, segs, srcs, rows, acc, gsems, ssems, zsem):
    cid = lax.axis_index("c")
    sid = lax.axis_index("s")
    nbase = sid * NNZ_PER_W

    _stage = [
        pltpu.make_async_copy(bidx_hbm.at[pl.ds(nbase, NNZ_PER_W)], tmpb, zsem),
        pltpu.make_async_copy(nidx_hbm.at[pl.ds(nbase, NNZ_PER_W)], tmpn, zsem),
        pltpu.make_async_copy(sidx_hbm.at[pl.ds(nbase, NNZ_PER_W)], tmps, zsem),
        pltpu.make_async_copy(vals_hbm.at[pl.ds(nbase, NNZ_PER_W)], vals_v, zsem),
    ]
    for d in _stage:
        d.start()
    for d in _stage:
        d.wait()

    @pl.loop(0, B_CHUNK)
    def _(r):
        for c in range(W // L):
            rows[0].at[r, pl.ds(c * L, L)][...] = jnp.zeros((L,), jnp.float32)

    @pl.loop(0, OUT_PER_W // B_CHUNK)
    def _(z):
        pltpu.async_copy(
            rows[0], acc.at[pl.ds(sid * OUT_PER_W + z * B_CHUNK, B_CHUNK)], zsem)

    @pl.loop(0, OUT_PER_W // B_CHUNK)
    def _(z):
        pltpu.make_async_copy(
            rows[0], acc.at[pl.ds(sid * OUT_PER_W + z * B_CHUNK, B_CHUNK)],
            zsem).wait()

    plsc.subcore_barrier()

    def compute_idx(i, e):
        st = i // SPC
        k = i % SPC
        for c in range(B_CHUNK // L):
            sl = pl.ds(st * B_CHUNK + c * L, L)
            dst = pl.ds(c * L, L)
            bb = tmpb.at[sl][...]
            segs[e].at[dst][...] = bb * N + tmpn.at[sl][...] + k * SEG
            srcs[e].at[dst][...] = (
                bb * S + tmps.at[sl][...] + (cid * SPC + k) * BS)

    def issue_gather(e):
        pltpu.async_copy(flat_hbm.at[srcs[e]], rows[e], gsems[e])

    def wait_gather(e):
        pltpu.make_async_copy(flat_hbm.at[srcs[e]], rows[e], gsems[e]).wait()

    def scale(i, e):
        st = i // SPC

        @pl.loop(0, B_CHUNK)
        def _(r):
            vb = plsc.load_gather(
                vals_v, [jnp.full((L,), st * B_CHUNK + r, jnp.int32)])
            for c in range(W // L):
                sl = (r, pl.ds(c * L, L))
                rows[e].at[*sl][...] = rows[e].at[*sl][...] * vb

    def issue_scatter(e):
        pltpu.async_copy(rows[e], acc.at[segs[e]], ssems[e], add=True)

    def wait_scatter(e):
        pltpu.make_async_copy(rows[e], acc.at[segs[e]], ssems[e]).wait()

    compute_idx(0, 0)
    issue_gather(0)

    @pl.loop(0, B_ITEMS // 2)
    def _(p):
        i0 = p * 2

        @pl.when(p > 0)
        def _():
            wait_scatter(1)

        compute_idx(i0 + 1, 1)
        issue_gather(1)
        wait_gather(0)
        scale(i0, 0)
        issue_scatter(0)

        @pl.when(p + 1 < B_ITEMS // 2)
        def _():
            wait_scatter(0)
            compute_idx(i0 + 2, 0)
            issue_gather(0)

        wait_gather(1)
        scale(i0 + 1, 1)
        issue_scatter(1)

    wait_scatter(0)
    wait_scatter(1)

    plsc.subcore_barrier()

    @pl.loop(0, OUT_PER_W // B_CHUNK)
    def _(z):
        ro = sid * OUT_PER_W + z * B_CHUNK
        pltpu.async_copy(acc.at[pl.ds(ro, B_CHUNK)],
                         out_hbm.at[pl.ds(cid * ACC_ROWS + ro, B_CHUNK)], zsem)

    @pl.loop(0, OUT_PER_W // B_CHUNK)
    def _(z):
        ro = sid * OUT_PER_W + z * B_CHUNK
        pltpu.make_async_copy(
            acc.at[pl.ds(ro, B_CHUNK)],
            out_hbm.at[pl.ds(cid * ACC_ROWS + ro, B_CHUNK)], zsem).wait()


def _ptt_body(p_ref, t_ref, o_ref):
    for t in range(TYPES):
        o_ref[t * MAXPOS:(t + 1) * MAXPOS, :] = p_ref[...] + t_ref[t:t + 1, :]


_LN_R = 512


def _ln_body(x_ref, g_ref, b_ref, o_ref):
    xs = [x_ref[k] for k in range(STRIPS)]
    s1 = sum(jnp.sum(x, axis=-1, keepdims=True) for x in xs)
    s2 = sum(jnp.sum(x * x, axis=-1, keepdims=True) for x in xs)
    mu = s1 * (1.0 / H)
    var = s2 * (1.0 / H) - mu * mu
    inv = lax.rsqrt(var + EPS)
    for k in range(STRIPS):
        o_ref[:, k * W:(k + 1) * W] = (
            (xs[k] - mu) * inv * g_ref[0, k * W:(k + 1) * W]
            + b_ref[0, k * W:(k + 1) * W])


def kernel(input_ids, mask_indices, mask_values, position_ids, token_type_ids,
           word_emb, pos_emb, type_emb, ln_gamma, ln_beta):
    ids = jnp.asarray(input_ids, jnp.int32).reshape(BS)
    pos = jnp.asarray(position_ids, jnp.int32).reshape(BS)
    tts = jnp.asarray(token_type_ids, jnp.int32).reshape(BS)
    mask = jnp.asarray(mask_indices, jnp.int32)

    ptt = pl.pallas_call(
        _ptt_body,
        out_shape=jax.ShapeDtypeStruct((TYPES * MAXPOS, H), jnp.float32),
    )(pos_emb, type_emb)

    flat = _phase_a(ids, pos, tts, word_emb, ptt)
    node6 = _phase_b(flat, mask[0], mask[1], mask[2], mask_values)

    out = pl.pallas_call(
        _ln_body,
        grid=(SEG // _LN_R,),
        in_specs=[
            pl.BlockSpec((STRIPS, _LN_R, W), lambda i: (0, i, 0)),
            pl.BlockSpec((1, H), lambda i: (0, 0)),
            pl.BlockSpec((1, H), lambda i: (0, 0)),
        ],
        out_specs=pl.BlockSpec((_LN_R, H), lambda i: (i, 0)),
        out_shape=jax.ShapeDtypeStruct((SEG, H), jnp.float32),
    )(node6.reshape(STRIPS, SEG, W), ln_gamma.reshape(1, H), ln_beta.reshape(1, H))

    return out.reshape(B, N, H)


# speedup vs baseline: 3.7325x; 1.0077x over previous
import dataclasses
import functools

import jax
import jax.numpy as jnp
from jax import lax
from jax.experimental import pallas as pl
from jax.experimental.pallas import tpu as pltpu
from jax.experimental.pallas import tpu_sc as plsc

NC, NS, L = 2, 16, 16
NW = NC * NS
B = 16
S = 512
N = 256
H = 768
MAXPOS = 512
TYPES = 2
W = 128
STRIPS = H // W
SPC = STRIPS // NC
BS = B * S
SEG = B * N
NNZ = 16384
EPS = 1e-12

ROWS_PER_W = BS // NW
A_CHUNK = 32
A_STEPS = ROWS_PER_W // A_CHUNK

NNZ_PER_W = NNZ // NS
B_CHUNK = 64
B_STEPS = NNZ_PER_W // B_CHUNK
B_ITEMS = B_STEPS * SPC
ACC_ROWS = SPC * SEG
OUT_PER_W = ACC_ROWS // NS

_mesh = plsc.VectorSubcoreMesh(
    core_axis_name="c", subcore_axis_name="s", num_cores=NC, num_subcores=NS
)

_cp = pltpu.CompilerParams()
if "needs_layout_passes" in pltpu.CompilerParams.__dataclass_fields__:
    _cp = dataclasses.replace(_cp, needs_layout_passes=False)


@functools.partial(
    pl.kernel,
    out_type=jax.ShapeDtypeStruct((STRIPS * BS, W), jnp.float32),
    mesh=_mesh,
    scratch_types=[
        pltpu.VMEM((ROWS_PER_W,), jnp.int32),
        pltpu.VMEM((ROWS_PER_W,), jnp.int32),
        pltpu.VMEM((ROWS_PER_W,), jnp.int32),
        [pltpu.VMEM((A_CHUNK, H), jnp.float32) for _ in range(2)],
        [pltpu.VMEM((A_CHUNK, H), jnp.float32) for _ in range(2)],
        [pltpu.SemaphoreType.DMA for _ in range(2)],
        [pltpu.SemaphoreType.DMA for _ in range(2)],
    ],
    compiler_params=_cp,
)
def _phase_a(ids_hbm, pos_hbm, tt_hbm, wtab, pttab, flat_out,
             widx, pidx, tidx, wrows, prows, gsems, wsems):
    wid = lax.axis_index("s") * NC + lax.axis_index("c")
    base = wid * ROWS_PER_W

    _stage = [
        pltpu.make_async_copy(ids_hbm.at[pl.ds(base, ROWS_PER_W)], widx, wsems[0]),
        pltpu.make_async_copy(pos_hbm.at[pl.ds(base, ROWS_PER_W)], pidx, wsems[0]),
        pltpu.make_async_copy(tt_hbm.at[pl.ds(base, ROWS_PER_W)], tidx, wsems[0]),
    ]
    for d in _stage:
        d.start()
    for d in _stage:
        d.wait()

    @pl.loop(0, ROWS_PER_W // L)
    def _(c):
        sl = pl.ds(c * L, L)
        pidx.at[sl][...] = pidx.at[sl][...] + tidx.at[sl][...] * MAXPOS

    def issue_gathers(st, e):
        sl = pl.ds(st * A_CHUNK, A_CHUNK)
        pltpu.async_copy(wtab.at[widx.at[sl]], wrows[e], gsems[e])
        pltpu.async_copy(pttab.at[pidx.at[sl]], prows[e], gsems[e])

    def wait_gathers(st, e):
        sl = pl.ds(st * A_CHUNK, A_CHUNK)
        pltpu.make_async_copy(wtab.at[widx.at[sl]], wrows[e], gsems[e]).wait()
        pltpu.make_async_copy(pttab.at[pidx.at[sl]], prows[e], gsems[e]).wait()

    def compute(st, e):
        @pl.loop(0, A_CHUNK)
        def _(r):
            for c in range(H // L):
                sl = (r, pl.ds(c * L, L))
                wrows[e].at[*sl][...] = (
                    wrows[e].at[*sl][...] + prows[e].at[*sl][...])

    def _write_descs(st, e):
        off = base + st * A_CHUNK
        for k in range(STRIPS):
            yield pltpu.make_async_copy(
                wrows[e].at[pl.ds(0, A_CHUNK), pl.ds(k * W, W)],
                flat_out.at[pl.ds(k * BS + off, A_CHUNK)], wsems[e])

    def issue_writes(st, e):
        for d in _write_descs(st, e):
            d.start()

    def wait_writes(st, e):
        for d in _write_descs(st, e):
            d.wait()

    issue_gathers(0, 0)

    @pl.loop(0, A_STEPS // 2)
    def _(p):
        st0 = p * 2

        @pl.when(p > 0)
        def _():
            wait_writes(st0 - 1, 1)

        issue_gathers(st0 + 1, 1)
        wait_gathers(st0, 0)
        compute(st0, 0)
        issue_writes(st0, 0)

        @pl.when(p + 1 < A_STEPS // 2)
        def _():
            wait_writes(st0, 0)
            issue_gathers(st0 + 2, 0)

        wait_gathers(st0 + 1, 1)
        compute(st0 + 1, 1)
        issue_writes(st0 + 1, 1)

    wait_writes(A_STEPS - 2, 0)
    wait_writes(A_STEPS - 1, 1)


@functools.partial(
    pl.kernel,
    out_type=jax.ShapeDtypeStruct((NC * ACC_ROWS, W), jnp.float32),
    mesh=_mesh,
    scratch_types=[
        pltpu.VMEM((NNZ_PER_W,), jnp.int32),
        pltpu.VMEM((NNZ_PER_W,), jnp.int32),
        pltpu.VMEM((NNZ_PER_W,), jnp.int32),
        pltpu.VMEM((NNZ_PER_W,), jnp.float32),
        [pltpu.VMEM((B_CHUNK,), jnp.int32) for _ in range(2)],
        [pltpu.VMEM((B_CHUNK,), jnp.int32) for _ in range(2)],
        [pltpu.VMEM((B_CHUNK, W), jnp.float32) for _ in range(2)],
        pltpu.VMEM_SHARED((ACC_ROWS, W), jnp.float32),
        [pltpu.SemaphoreType.DMA for _ in range(2)],
        [pltpu.SemaphoreType.DMA for _ in range(2)],
        pltpu.SemaphoreType.DMA,
    ],
    compiler_params=_cp,
)
def _phase_b(flat_hbm, bidx_hbm, nidx_hbm, sidx_hbm, vals_hbm, out_hbm,
             tmpb, tmpn, tmps, vals_v, segs, srcs, rows, acc, gsems, ssems, zsem):
    cid = lax.axis_index("c")
    sid = lax.axis_index("s")
    nbase = sid * NNZ_PER_W

    _stage = [
        pltpu.make_async_copy(bidx_hbm.at[pl.ds(nbase, NNZ_PER_W)], tmpb, zsem),
        pltpu.make_async_copy(nidx_hbm.at[pl.ds(nbase, NNZ_PER_W)], tmpn, zsem),
        pltpu.make_async_copy(sidx_hbm.at[pl.ds(nbase, NNZ_PER_W)], tmps, zsem),
        pltpu.make_async_copy(vals_hbm.at[pl.ds(nbase, NNZ_PER_W)], vals_v, zsem),
    ]
    for d in _stage:
        d.start()

    @pl.loop(0, B_CHUNK)
    def _(r):
        for c in range(W // L):
            rows[0].at[r, pl.ds(c * L, L)][...] = jnp.zeros((L,), jnp.float32)

    @pl.loop(0, OUT_PER_W // B_CHUNK)
    def _(z):
        pltpu.async_copy(
            rows[0], acc.at[pl.ds(sid * OUT_PER_W + z * B_CHUNK, B_CHUNK)], zsem)

    for d in _stage:
        d.wait()

    @pl.loop(0, OUT_PER_W // B_CHUNK)
    def _(z):
        pltpu.make_async_copy(
            rows[0], acc.at[pl.ds(sid * OUT_PER_W + z * B_CHUNK, B_CHUNK)],
            zsem).wait()

    def compute_idx(i, e):
        st = i // SPC
        k = i % SPC
        for c in range(B_CHUNK // L):
            sl = pl.ds(st * B_CHUNK + c * L, L)
            dst = pl.ds(c * L, L)
            bb = tmpb.at[sl][...]
            segs[e].at[dst][...] = bb * N + tmpn.at[sl][...] + k * SEG
            srcs[e].at[dst][...] = (
                bb * S + tmps.at[sl][...] + (cid * SPC + k) * BS)

    def issue_gather(e):
        pltpu.async_copy(flat_hbm.at[srcs[e]], rows[e], gsems[e])

    def wait_gather(e):
        pltpu.make_async_copy(flat_hbm.at[srcs[e]], rows[e], gsems[e]).wait()

    def scale(i, e):
        st = i // SPC

        @pl.loop(0, B_CHUNK)
        def _(r):
            vb = plsc.load_gather(
                vals_v, [jnp.full((L,), st * B_CHUNK + r, jnp.int32)])
            for c in range(W // L):
                sl = (r, pl.ds(c * L, L))
                rows[e].at[*sl][...] = rows[e].at[*sl][...] * vb

    def issue_scatter(e):
        pltpu.async_copy(rows[e], acc.at[segs[e]], ssems[e], add=True)

    def wait_scatter(e):
        pltpu.make_async_copy(rows[e], acc.at[segs[e]], ssems[e]).wait()

    compute_idx(0, 0)
    issue_gather(0)

    plsc.subcore_barrier()

    @pl.loop(0, B_ITEMS // 2)
    def _(p):
        i0 = p * 2

        @pl.when(p > 0)
        def _():
            wait_scatter(1)

        compute_idx(i0 + 1, 1)
        issue_gather(1)
        wait_gather(0)
        scale(i0, 0)
        issue_scatter(0)

        @pl.when(p + 1 < B_ITEMS // 2)
        def _():
            wait_scatter(0)
            compute_idx(i0 + 2, 0)
            issue_gather(0)

        wait_gather(1)
        scale(i0 + 1, 1)
        issue_scatter(1)

    wait_scatter(0)
    wait_scatter(1)

    plsc.subcore_barrier()

    ro = sid * OUT_PER_W
    pltpu.sync_copy(acc.at[pl.ds(ro, OUT_PER_W)],
                    out_hbm.at[pl.ds(cid * ACC_ROWS + ro, OUT_PER_W)])


def _ptt_body(p_ref, t_ref, o_ref):
    for t in range(TYPES):
        o_ref[t * MAXPOS:(t + 1) * MAXPOS, :] = p_ref[...] + t_ref[t:t + 1, :]


_LN_R = 512


def _ln_body(x_ref, g_ref, b_ref, o_ref):
    xs = [x_ref[k] for k in range(STRIPS)]
    s1 = sum(jnp.sum(x, axis=-1, keepdims=True) for x in xs)
    s2 = sum(jnp.sum(x * x, axis=-1, keepdims=True) for x in xs)
    mu = s1 * (1.0 / H)
    var = s2 * (1.0 / H) - mu * mu
    inv = lax.rsqrt(var + EPS)
    for k in range(STRIPS):
        o_ref[:, k * W:(k + 1) * W] = (
            (xs[k] - mu) * inv * g_ref[0, k * W:(k + 1) * W]
            + b_ref[0, k * W:(k + 1) * W])


def kernel(input_ids, mask_indices, mask_values, position_ids, token_type_ids,
           word_emb, pos_emb, type_emb, ln_gamma, ln_beta):
    ids = jnp.asarray(input_ids, jnp.int32).reshape(BS)
    pos = jnp.asarray(position_ids, jnp.int32).reshape(BS)
    tts = jnp.asarray(token_type_ids, jnp.int32).reshape(BS)
    mask = jnp.asarray(mask_indices, jnp.int32)

    ptt = pl.pallas_call(
        _ptt_body,
        out_shape=jax.ShapeDtypeStruct((TYPES * MAXPOS, H), jnp.float32),
    )(pos_emb, type_emb)

    flat = _phase_a(ids, pos, tts, word_emb, ptt)
    node6 = _phase_b(flat, mask[0], mask[1], mask[2], mask_values)

    out = pl.pallas_call(
        _ln_body,
        grid=(SEG // _LN_R,),
        in_specs=[
            pl.BlockSpec((STRIPS, _LN_R, W), lambda i: (0, i, 0)),
            pl.BlockSpec((1, H), lambda i: (0, 0)),
            pl.BlockSpec((1, H), lambda i: (0, 0)),
        ],
        out_specs=pl.BlockSpec((_LN_R, H), lambda i: (i, 0)),
        out_shape=jax.ShapeDtypeStruct((SEG, H), jnp.float32),
    )(node6.reshape(STRIPS, SEG, W), ln_gamma.reshape(1, H), ln_beta.reshape(1, H))

    return out.reshape(B, N, H)


# speedup vs baseline: 3.9719x; 1.0641x over previous
import dataclasses
import functools

import jax
import jax.numpy as jnp
from jax import lax
from jax.experimental import pallas as pl
from jax.experimental.pallas import tpu as pltpu
from jax.experimental.pallas import tpu_sc as plsc

NC, NS, L = 2, 16, 16
NW = NC * NS
B = 16
S = 512
N = 256
H = 768
MAXPOS = 512
TYPES = 2
W = 128
STRIPS = H // W
SPC = STRIPS // NC
BS = B * S
SEG = B * N
NNZ = 16384
EPS = 1e-12

ROWS_PER_W = BS // NW
A_CHUNK = 32
A_STEPS = ROWS_PER_W // A_CHUNK

NNZ_PER_W = NNZ // NS
B_CHUNK = 64
B_STEPS = NNZ_PER_W // B_CHUNK
B_ITEMS = B_STEPS * SPC
ACC_ROWS = SPC * SEG
OUT_PER_W = ACC_ROWS // NS

_mesh = plsc.VectorSubcoreMesh(
    core_axis_name="c", subcore_axis_name="s", num_cores=NC, num_subcores=NS
)

_cp = pltpu.CompilerParams()
if "needs_layout_passes" in pltpu.CompilerParams.__dataclass_fields__:
    _cp = dataclasses.replace(_cp, needs_layout_passes=False)


@functools.partial(
    pl.kernel,
    out_type=jax.ShapeDtypeStruct((STRIPS * BS, W), jnp.float32),
    mesh=_mesh,
    scratch_types=[
        pltpu.VMEM((ROWS_PER_W,), jnp.int32),
        pltpu.VMEM((ROWS_PER_W,), jnp.int32),
        pltpu.VMEM((ROWS_PER_W,), jnp.int32),
        [pltpu.VMEM((A_CHUNK, H), jnp.float32) for _ in range(2)],
        [pltpu.VMEM((A_CHUNK, H), jnp.float32) for _ in range(2)],
        [pltpu.SemaphoreType.DMA for _ in range(2)],
        [pltpu.SemaphoreType.DMA for _ in range(2)],
    ],
    compiler_params=_cp,
)
def _phase_a(ids_hbm, pos_hbm, tt_hbm, wtab, pttab, flat_out,
             widx, pidx, tidx, wrows, prows, gsems, wsems):
    wid = lax.axis_index("s") * NC + lax.axis_index("c")
    base = wid * ROWS_PER_W

    _stage = [
        pltpu.make_async_copy(ids_hbm.at[pl.ds(base, ROWS_PER_W)], widx, wsems[0]),
        pltpu.make_async_copy(pos_hbm.at[pl.ds(base, ROWS_PER_W)], pidx, wsems[0]),
        pltpu.make_async_copy(tt_hbm.at[pl.ds(base, ROWS_PER_W)], tidx, wsems[0]),
    ]
    for d in _stage:
        d.start()
    for d in _stage:
        d.wait()

    @pl.loop(0, ROWS_PER_W // L)
    def _(c):
        sl = pl.ds(c * L, L)
        pidx.at[sl][...] = pidx.at[sl][...] + tidx.at[sl][...] * MAXPOS

    def issue_gathers(st, e):
        sl = pl.ds(st * A_CHUNK, A_CHUNK)
        pltpu.async_copy(wtab.at[widx.at[sl]], wrows[e], gsems[e])
        pltpu.async_copy(pttab.at[pidx.at[sl]], prows[e], gsems[e])

    def wait_gathers(st, e):
        sl = pl.ds(st * A_CHUNK, A_CHUNK)
        pltpu.make_async_copy(wtab.at[widx.at[sl]], wrows[e], gsems[e]).wait()
        pltpu.make_async_copy(pttab.at[pidx.at[sl]], prows[e], gsems[e]).wait()

    def compute(st, e):
        @plsc.parallel_loop(0, A_CHUNK, unroll=2)
        def _(r):
            for c in range(H // L):
                sl = (r, pl.ds(c * L, L))
                wrows[e].at[*sl][...] = (
                    wrows[e].at[*sl][...] + prows[e].at[*sl][...])

    def _write_descs(st, e):
        off = base + st * A_CHUNK
        for k in range(STRIPS):
            yield pltpu.make_async_copy(
                wrows[e].at[pl.ds(0, A_CHUNK), pl.ds(k * W, W)],
                flat_out.at[pl.ds(k * BS + off, A_CHUNK)], wsems[e])

    def issue_writes(st, e):
        for d in _write_descs(st, e):
            d.start()

    def wait_writes(st, e):
        for d in _write_descs(st, e):
            d.wait()

    issue_gathers(0, 0)

    @pl.loop(0, A_STEPS // 2)
    def _(p):
        st0 = p * 2

        @pl.when(p > 0)
        def _():
            wait_writes(st0 - 1, 1)

        issue_gathers(st0 + 1, 1)
        wait_gathers(st0, 0)
        compute(st0, 0)
        issue_writes(st0, 0)

        @pl.when(p + 1 < A_STEPS // 2)
        def _():
            wait_writes(st0, 0)
            issue_gathers(st0 + 2, 0)

        wait_gathers(st0 + 1, 1)
        compute(st0 + 1, 1)
        issue_writes(st0 + 1, 1)

    wait_writes(A_STEPS - 2, 0)
    wait_writes(A_STEPS - 1, 1)


@functools.partial(
    pl.kernel,
    out_type=jax.ShapeDtypeStruct((NC * ACC_ROWS, W), jnp.float32),
    mesh=_mesh,
    scratch_types=[
        pltpu.VMEM((NNZ_PER_W,), jnp.int32),
        pltpu.VMEM((NNZ_PER_W,), jnp.int32),
        pltpu.VMEM((NNZ_PER_W,), jnp.int32),
        pltpu.VMEM((NNZ_PER_W,), jnp.float32),
        [pltpu.VMEM((B_CHUNK,), jnp.int32) for _ in range(2)],
        [pltpu.VMEM((B_CHUNK,), jnp.int32) for _ in range(2)],
        [pltpu.VMEM((B_CHUNK, W), jnp.float32) for _ in range(2)],
        pltpu.VMEM_SHARED((ACC_ROWS, W), jnp.float32),
        [pltpu.SemaphoreType.DMA for _ in range(2)],
        [pltpu.SemaphoreType.DMA for _ in range(2)],
        pltpu.SemaphoreType.DMA,
    ],
    compiler_params=_cp,
)
def _phase_b(flat_hbm, bidx_hbm, nidx_hbm, sidx_hbm, vals_hbm, out_hbm,
             tmpb, tmpn, tmps, vals_v, segs, srcs, rows, acc, gsems, ssems, zsem):
    cid = lax.axis_index("c")
    sid = lax.axis_index("s")
    nbase = sid * NNZ_PER_W

    _stage = [
        pltpu.make_async_copy(bidx_hbm.at[pl.ds(nbase, NNZ_PER_W)], tmpb, zsem),
        pltpu.make_async_copy(nidx_hbm.at[pl.ds(nbase, NNZ_PER_W)], tmpn, zsem),
        pltpu.make_async_copy(sidx_hbm.at[pl.ds(nbase, NNZ_PER_W)], tmps, zsem),
        pltpu.make_async_copy(vals_hbm.at[pl.ds(nbase, NNZ_PER_W)], vals_v, zsem),
    ]
    for d in _stage:
        d.start()

    @pl.loop(0, B_CHUNK)
    def _(r):
        for c in range(W // L):
            rows[0].at[r, pl.ds(c * L, L)][...] = jnp.zeros((L,), jnp.float32)

    @pl.loop(0, OUT_PER_W // B_CHUNK)
    def _(z):
        pltpu.async_copy(
            rows[0], acc.at[pl.ds(sid * OUT_PER_W + z * B_CHUNK, B_CHUNK)], zsem)

    for d in _stage:
        d.wait()

    @pl.loop(0, OUT_PER_W // B_CHUNK)
    def _(z):
        pltpu.make_async_copy(
            rows[0], acc.at[pl.ds(sid * OUT_PER_W + z * B_CHUNK, B_CHUNK)],
            zsem).wait()

    def compute_idx(i, e):
        st = i // SPC
        k = i % SPC
        for c in range(B_CHUNK // L):
            sl = pl.ds(st * B_CHUNK + c * L, L)
            dst = pl.ds(c * L, L)
            bb = tmpb.at[sl][...]
            segs[e].at[dst][...] = bb * N + tmpn.at[sl][...] + k * SEG
            srcs[e].at[dst][...] = (
                bb * S + tmps.at[sl][...] + (cid * SPC + k) * BS)

    def issue_gather(e):
        pltpu.async_copy(flat_hbm.at[srcs[e]], rows[e], gsems[e])

    def wait_gather(e):
        pltpu.make_async_copy(flat_hbm.at[srcs[e]], rows[e], gsems[e]).wait()

    def scale(i, e):
        st = i // SPC

        @plsc.parallel_loop(0, B_CHUNK, unroll=4)
        def _(r):
            vb = plsc.load_gather(
                vals_v, [jnp.full((L,), st * B_CHUNK + r, jnp.int32)])
            for c in range(W // L):
                sl = (r, pl.ds(c * L, L))
                rows[e].at[*sl][...] = rows[e].at[*sl][...] * vb

    def issue_scatter(e):
        pltpu.async_copy(rows[e], acc.at[segs[e]], ssems[e], add=True)

    def wait_scatter(e):
        pltpu.make_async_copy(rows[e], acc.at[segs[e]], ssems[e]).wait()

    compute_idx(0, 0)
    issue_gather(0)

    plsc.subcore_barrier()

    @pl.loop(0, B_ITEMS // 2)
    def _(p):
        i0 = p * 2

        @pl.when(p > 0)
        def _():
            wait_scatter(1)

        compute_idx(i0 + 1, 1)
        issue_gather(1)
        wait_gather(0)
        scale(i0, 0)
        issue_scatter(0)

        @pl.when(p + 1 < B_ITEMS // 2)
        def _():
            wait_scatter(0)
            compute_idx(i0 + 2, 0)
            issue_gather(0)

        wait_gather(1)
        scale(i0 + 1, 1)
        issue_scatter(1)

    wait_scatter(0)
    wait_scatter(1)

    plsc.subcore_barrier()

    ro = sid * OUT_PER_W
    pltpu.sync_copy(acc.at[pl.ds(ro, OUT_PER_W)],
                    out_hbm.at[pl.ds(cid * ACC_ROWS + ro, OUT_PER_W)])


def _ptt_body(p_ref, t_ref, o_ref):
    for t in range(TYPES):
        o_ref[t * MAXPOS:(t + 1) * MAXPOS, :] = p_ref[...] + t_ref[t:t + 1, :]


_LN_R = 512


def _ln_body(x_ref, g_ref, b_ref, o_ref):
    xs = [x_ref[k] for k in range(STRIPS)]
    s1 = sum(jnp.sum(x, axis=-1, keepdims=True) for x in xs)
    s2 = sum(jnp.sum(x * x, axis=-1, keepdims=True) for x in xs)
    mu = s1 * (1.0 / H)
    var = s2 * (1.0 / H) - mu * mu
    inv = lax.rsqrt(var + EPS)
    for k in range(STRIPS):
        o_ref[:, k * W:(k + 1) * W] = (
            (xs[k] - mu) * inv * g_ref[0, k * W:(k + 1) * W]
            + b_ref[0, k * W:(k + 1) * W])


def kernel(input_ids, mask_indices, mask_values, position_ids, token_type_ids,
           word_emb, pos_emb, type_emb, ln_gamma, ln_beta):
    ids = jnp.asarray(input_ids, jnp.int32).reshape(BS)
    pos = jnp.asarray(position_ids, jnp.int32).reshape(BS)
    tts = jnp.asarray(token_type_ids, jnp.int32).reshape(BS)
    mask = jnp.asarray(mask_indices, jnp.int32)

    ptt = pl.pallas_call(
        _ptt_body,
        out_shape=jax.ShapeDtypeStruct((TYPES * MAXPOS, H), jnp.float32),
    )(pos_emb, type_emb)

    flat = _phase_a(ids, pos, tts, word_emb, ptt)
    node6 = _phase_b(flat, mask[0], mask[1], mask[2], mask_values)

    out = pl.pallas_call(
        _ln_body,
        grid=(SEG // _LN_R,),
        in_specs=[
            pl.BlockSpec((STRIPS, _LN_R, W), lambda i: (0, i, 0)),
            pl.BlockSpec((1, H), lambda i: (0, 0)),
            pl.BlockSpec((1, H), lambda i: (0, 0)),
        ],
        out_specs=pl.BlockSpec((_LN_R, H), lambda i: (i, 0)),
        out_shape=jax.ShapeDtypeStruct((SEG, H), jnp.float32),
    )(node6.reshape(STRIPS, SEG, W), ln_gamma.reshape(1, H), ln_beta.reshape(1, H))

    return out.reshape(B, N, H)
